# baseline (speedup 1.0000x reference)
import functools

import jax
import jax.numpy as jnp
import numpy as np
from jax import lax
from jax.experimental import pallas as pl
from jax.experimental.pallas import tpu as pltpu
from jax.experimental.pallas import tpu_sc as plsc

B = 8
NGT = 50
DN = 100
TOTAL = B * DN * NGT
PERB = DN * NGT
HID = 256
BTOT = 4 * TOTAL

NW = 32
SHARE = 1248
QPAD = 1280
CHUNK = 128
NCH = 10
LAST = SHARE - 9 * CHUNK
BELEM = 4 * (SHARE + 8)
LDBL = 72
BDBL = 400

_STARTS = np.array([(w // 4) * PERB + (w % 4) * SHARE for w in range(NW)])
_PACK_IDX = np.minimum(_STARTS[:, None] + np.arange(QPAD)[None, :],
                       TOTAL - 1).astype(np.int32).reshape(-1)
_UNPACK_IDX = np.concatenate([
    w * QPAD + np.arange(SHARE + (8 if w % 4 == 3 else 0))
    for w in range(NW)
]).astype(np.int32)


@functools.cache
def _build_sc():
    mesh = plsc.VectorSubcoreMesh(core_axis_name="c", subcore_axis_name="s")
    return pl.kernel(
        _sc_body,
        mesh=mesh,
        out_type=(
            jax.ShapeDtypeStruct((TOTAL, HID), jnp.float32),
            jax.ShapeDtypeStruct((NW * QPAD,), jnp.int32),
            jax.ShapeDtypeStruct((BTOT,), jnp.float32),
            jax.ShapeDtypeStruct((BTOT,), jnp.float32),
        ),
        scratch_types=[
            pltpu.VMEM((LDBL,), jnp.int32),
            pltpu.VMEM((BDBL,), jnp.float32),
            pltpu.VMEM((BDBL,), jnp.float32),
            pltpu.VMEM((QPAD,), jnp.float32),
            pltpu.VMEM((QPAD,), jnp.int32),
            pltpu.VMEM((NCH, CHUNK), jnp.int32),
            pltpu.VMEM((QPAD,), jnp.int32),
            pltpu.VMEM((CHUNK, HID), jnp.float32),
            pltpu.VMEM((CHUNK, HID), jnp.float32),
            pltpu.VMEM((BELEM,), jnp.float32),
            pltpu.VMEM((BELEM,), jnp.float32),
            pltpu.VMEM((BELEM,), jnp.float32),
            pltpu.SemaphoreType.DMA,
            pltpu.SemaphoreType.DMA,
            pltpu.SemaphoreType.DMA,
            pltpu.SemaphoreType.DMA,
            pltpu.SemaphoreType.DMA,
        ],
    )


def _sc_body(ldbl_h, bdbl_h, bcomp_h, nu_h, rl_h, bnr_h, table_h,
             q_out, lab_out, ref_out, tbox_out,
             ldbl_v, bdbl_v, bcomp_v, nu_v, rl_v, idx_v, lab_v,
             rows0_v, rows1_v, bnr_v, refo_v, tbo_v,
             sem_in, sem_g0, sem_g1, sem_o0, sem_o1):
    wid = lax.axis_index("s") * 2 + lax.axis_index("c")
    b = wid // 4
    lw = lax.rem(wid, 4)
    qrow0 = b * PERB + lw * SHARE
    bbase = 4 * qrow0
    qphase0 = lax.rem(qrow0, NGT)
    bphase0 = lax.rem(bbase, 4 * NGT)
    is_big = lw == 3

    ins = [
        pltpu.async_copy(ldbl_h.at[pl.ds(b * LDBL, LDBL)], ldbl_v, sem_in),
        pltpu.async_copy(bdbl_h.at[pl.ds(b * BDBL, BDBL)], bdbl_v, sem_in),
        pltpu.async_copy(bcomp_h.at[pl.ds(b * BDBL, BDBL)], bcomp_v, sem_in),
        pltpu.async_copy(nu_h.at[pl.ds(wid * QPAD, QPAD)], nu_v, sem_in),
        pltpu.async_copy(rl_h.at[pl.ds(wid * QPAD, QPAD)], rl_v, sem_in),
        pltpu.async_copy(bnr_h.at[pl.ds(bbase, 4 * SHARE)],
                         bnr_v.at[pl.ds(0, 4 * SHARE)], sem_in),
    ]
    for c in ins:
        c.wait()

    @pl.when(is_big)
    def _():
        pltpu.sync_copy(bnr_h.at[pl.ds(bbase + 4 * SHARE, 32)],
                        bnr_v.at[pl.ds(4 * SHARE, 32)])

    rows = (rows0_v, rows1_v)
    gsem = (sem_g0, sem_g1)
    osem = (sem_o0, sem_o1)
    gh = [None, None]
    oh = [None, None]

    for j in range(NCH):
        def grp(k, carry, j=j):
            off = j * CHUNK + k * 16
            phase = lax.rem(qphase0 + off, NGT)
            rep = ldbl_v[pl.ds(phase, 16)]
            lab_v[pl.ds(off, 16)] = rep
            nu = nu_v[pl.ds(off, 16)]
            rl = rl_v[pl.ds(off, 16)]
            idx_v[j, pl.ds(k * 16, 16)] = jnp.where(nu < 0.5, rl, rep)
            return carry
        lax.fori_loop(0, CHUNK // 16, grp, 0)
        s = j & 1
        if oh[s] is not None:
            oh[s].wait()
            oh[s] = None
        gh[s] = pltpu.async_copy(table_h.at[idx_v.at[j]], rows[s], gsem[s])
        if j >= 1:
            p = (j - 1) & 1
            gh[p].wait()
            gh[p] = None
            n = CHUNK if j - 1 < NCH - 1 else LAST
            oh[p] = pltpu.async_copy(
                rows[p].at[pl.ds(0, n)],
                q_out.at[pl.ds(qrow0 + (j - 1) * CHUNK, n)], osem[p])

    s = (NCH - 1) & 1
    gh[s].wait()
    oh[s] = pltpu.async_copy(
        rows[s].at[pl.ds(0, LAST)],
        q_out.at[pl.ds(qrow0 + (NCH - 1) * CHUNK, LAST)], osem[s])

    @pl.when(is_big)
    def _():
        pltpu.sync_copy(
            rows[s].at[pl.ds(LAST, 8)],
            q_out.at[pl.ds(qrow0 + (NCH - 1) * CHUNK + LAST, 8)])

    def bgrp(i, carry):
        off = i * 16
        bphase = lax.rem(bphase0 + off, 4 * NGT)
        rb = bdbl_v[pl.ds(bphase, 16)]
        cb = bcomp_v[pl.ds(bphase, 16)]
        bn = bnr_v[pl.ds(off, 16)] * 0.8 - 0.4
        out = jnp.minimum(jnp.maximum(rb + bn * cb, 0.0), 1.0)
        refo_v[pl.ds(off, 16)] = out
        tbo_v[pl.ds(off, 16)] = rb
        return carry
    lax.fori_loop(0, BELEM // 16, bgrp, 0)

    pltpu.sync_copy(lab_v, lab_out.at[pl.ds(wid * QPAD, QPAD)])
    pltpu.sync_copy(refo_v.at[pl.ds(0, 4 * SHARE)],
                    ref_out.at[pl.ds(bbase, 4 * SHARE)])
    pltpu.sync_copy(tbo_v.at[pl.ds(0, 4 * SHARE)],
                    tbox_out.at[pl.ds(bbase, 4 * SHARE)])

    @pl.when(is_big)
    def _():
        pltpu.sync_copy(refo_v.at[pl.ds(4 * SHARE, 32)],
                        ref_out.at[pl.ds(bbase + 4 * SHARE, 32)])
        pltpu.sync_copy(tbo_v.at[pl.ds(4 * SHARE, 32)],
                        tbox_out.at[pl.ds(bbase + 4 * SHARE, 32)])

    for h in oh:
        if h is not None:
            h.wait()


def kernel(labels, boxes, noise_u, rand_labels, box_noise_raw, table):
    labels = labels.astype(jnp.int32)
    ldbl = jnp.concatenate([labels, labels[:, : LDBL - NGT]], axis=1)
    boxes_r = boxes.reshape(B, 4 * NGT)
    bdbl = jnp.concatenate([boxes_r, boxes_r[:, : BDBL - 4 * NGT]], axis=1)
    comp_r = boxes[:, :, (2, 3, 2, 3)].reshape(B, 4 * NGT)
    bcomp = jnp.concatenate([comp_r, comp_r[:, : BDBL - 4 * NGT]], axis=1)
    pack = jnp.asarray(_PACK_IDX)
    nu = noise_u.reshape(-1)[pack]
    rl = rand_labels.astype(jnp.int32).reshape(-1)[pack]
    q, lab, refo, tbo = _build_sc()(
        ldbl.reshape(-1), bdbl.reshape(-1), bcomp.reshape(-1),
        nu, rl, box_noise_raw.reshape(-1), table)
    dn_query = q.reshape(B, PERB, HID)
    dn_ref = refo.reshape(B, PERB, 4)
    dn_query_pos = jnp.zeros_like(dn_query)
    dn_target_labels = lab[jnp.asarray(_UNPACK_IDX)].reshape(B, PERB)
    dn_target_boxes = tbo.reshape(B, PERB, 4)
    return (dn_query, dn_ref, dn_query_pos, dn_target_labels, dn_target_boxes)


# speedup vs baseline: 1.1680x; 1.1680x over previous
import functools

import jax
import jax.numpy as jnp
from jax import lax
from jax.experimental import pallas as pl
from jax.experimental.pallas import tpu as pltpu
from jax.experimental.pallas import tpu_sc as plsc

B = 8
NGT = 50
DN = 100
TOTAL = B * DN * NGT
PERB = DN * NGT
HID = 256
BTOT = 4 * TOTAL

NW = 32
SHARE = 1248
QPAD = 1280
CHUNK = 128
NCH = 10
LAST = SHARE - 9 * CHUNK
BELEM = 4 * (SHARE + 8)
LDBL = 72
BDBL = 400

NUPAD = 40032


@functools.cache
def _build_sc():
    mesh = plsc.VectorSubcoreMesh(core_axis_name="c", subcore_axis_name="s")
    return pl.kernel(
        _sc_body,
        mesh=mesh,
        out_type=(
            jax.ShapeDtypeStruct((TOTAL, HID), jnp.float32),
            jax.ShapeDtypeStruct((TOTAL,), jnp.int32),
            jax.ShapeDtypeStruct((BTOT,), jnp.float32),
            jax.ShapeDtypeStruct((BTOT,), jnp.float32),
        ),
        scratch_types=[
            pltpu.VMEM((LDBL,), jnp.int32),
            pltpu.VMEM((BDBL,), jnp.float32),
            pltpu.VMEM((BDBL,), jnp.float32),
            pltpu.VMEM((QPAD,), jnp.float32),
            pltpu.VMEM((QPAD,), jnp.int32),
            pltpu.VMEM((NCH, CHUNK), jnp.int32),
            pltpu.VMEM((QPAD,), jnp.int32),
            pltpu.VMEM((CHUNK, HID), jnp.float32),
            pltpu.VMEM((CHUNK, HID), jnp.float32),
            pltpu.VMEM((BELEM,), jnp.float32),
            pltpu.VMEM((BELEM,), jnp.float32),
            pltpu.VMEM((BELEM,), jnp.float32),
            pltpu.SemaphoreType.DMA,
            pltpu.SemaphoreType.DMA,
            pltpu.SemaphoreType.DMA,
            pltpu.SemaphoreType.DMA,
            pltpu.SemaphoreType.DMA,
        ],
    )


def _sc_body(ldbl_h, bdbl_h, bcomp_h, nu_h, rl_h, bnr_h, table_h,
             q_out, lab_out, ref_out, tbox_out,
             ldbl_v, bdbl_v, bcomp_v, nu_v, rl_v, idx_v, lab_v,
             rows0_v, rows1_v, bnr_v, refo_v, tbo_v,
             sem_in, sem_g0, sem_g1, sem_o0, sem_o1):
    wid = lax.axis_index("s") * 2 + lax.axis_index("c")
    b = wid // 4
    lw = lax.rem(wid, 4)
    qrow0 = b * PERB + lw * SHARE
    bbase = 4 * qrow0
    qphase0 = lax.rem(qrow0, NGT)
    bphase0 = lax.rem(bbase, 4 * NGT)
    is_big = lw == 3

    ins = [
        pltpu.async_copy(ldbl_h.at[pl.ds(b * LDBL, LDBL)], ldbl_v, sem_in),
        pltpu.async_copy(bdbl_h.at[pl.ds(b * BDBL, BDBL)], bdbl_v, sem_in),
        pltpu.async_copy(bcomp_h.at[pl.ds(b * BDBL, BDBL)], bcomp_v, sem_in),
        pltpu.async_copy(nu_h.at[pl.ds(qrow0, QPAD)], nu_v, sem_in),
        pltpu.async_copy(rl_h.at[pl.ds(qrow0, QPAD)], rl_v, sem_in),
        pltpu.async_copy(bnr_h.at[pl.ds(bbase, 4 * SHARE)],
                         bnr_v.at[pl.ds(0, 4 * SHARE)], sem_in),
    ]
    for c in ins:
        c.wait()

    @pl.when(is_big)
    def _():
        pltpu.sync_copy(bnr_h.at[pl.ds(bbase + 4 * SHARE, 32)],
                        bnr_v.at[pl.ds(4 * SHARE, 32)])

    rows = (rows0_v, rows1_v)
    gsem = (sem_g0, sem_g1)
    osem = (sem_o0, sem_o1)
    gh = [None, None]
    oh = [None, None]

    for j in range(NCH):
        def grp(k, carry, j=j):
            off = j * CHUNK + k * 16
            phase = lax.rem(qphase0 + off, NGT)
            rep = ldbl_v[pl.ds(phase, 16)]
            lab_v[pl.ds(off, 16)] = rep
            nu = nu_v[pl.ds(off, 16)]
            rl = rl_v[pl.ds(off, 16)]
            idx_v[j, pl.ds(k * 16, 16)] = jnp.where(nu < 0.5, rl, rep)
            return carry
        lax.fori_loop(0, CHUNK // 16, grp, 0)
        s = j & 1
        if oh[s] is not None:
            oh[s].wait()
            oh[s] = None
        gh[s] = pltpu.async_copy(table_h.at[idx_v.at[j]], rows[s], gsem[s])
        if j >= 1:
            p = (j - 1) & 1
            gh[p].wait()
            gh[p] = None
            n = CHUNK if j - 1 < NCH - 1 else LAST
            oh[p] = pltpu.async_copy(
                rows[p].at[pl.ds(0, n)],
                q_out.at[pl.ds(qrow0 + (j - 1) * CHUNK, n)], osem[p])

    s = (NCH - 1) & 1
    gh[s].wait()
    oh[s] = pltpu.async_copy(
        rows[s].at[pl.ds(0, LAST)],
        q_out.at[pl.ds(qrow0 + (NCH - 1) * CHUNK, LAST)], osem[s])

    @pl.when(is_big)
    def _():
        pltpu.sync_copy(
            rows[s].at[pl.ds(LAST, 8)],
            q_out.at[pl.ds(qrow0 + (NCH - 1) * CHUNK + LAST, 8)])

    def bgrp(i, carry):
        off = i * 16
        bphase = lax.rem(bphase0 + off, 4 * NGT)
        rb = bdbl_v[pl.ds(bphase, 16)]
        cb = bcomp_v[pl.ds(bphase, 16)]
        bn = bnr_v[pl.ds(off, 16)] * 0.8 - 0.4
        out = jnp.minimum(jnp.maximum(rb + bn * cb, 0.0), 1.0)
        refo_v[pl.ds(off, 16)] = out
        tbo_v[pl.ds(off, 16)] = rb
        return carry
    lax.fori_loop(0, BELEM // 16, bgrp, 0)

    pltpu.sync_copy(lab_v.at[pl.ds(0, SHARE)],
                    lab_out.at[pl.ds(qrow0, SHARE)])
    pltpu.sync_copy(refo_v.at[pl.ds(0, 4 * SHARE)],
                    ref_out.at[pl.ds(bbase, 4 * SHARE)])
    pltpu.sync_copy(tbo_v.at[pl.ds(0, 4 * SHARE)],
                    tbox_out.at[pl.ds(bbase, 4 * SHARE)])

    @pl.when(is_big)
    def _():
        pltpu.sync_copy(lab_v.at[pl.ds(SHARE, 8)],
                        lab_out.at[pl.ds(qrow0 + SHARE, 8)])
        pltpu.sync_copy(refo_v.at[pl.ds(4 * SHARE, 32)],
                        ref_out.at[pl.ds(bbase + 4 * SHARE, 32)])
        pltpu.sync_copy(tbo_v.at[pl.ds(4 * SHARE, 32)],
                        tbox_out.at[pl.ds(bbase + 4 * SHARE, 32)])

    for h in oh:
        if h is not None:
            h.wait()


def kernel(labels, boxes, noise_u, rand_labels, box_noise_raw, table):
    labels = labels.astype(jnp.int32)
    ldbl = jnp.concatenate([labels, labels[:, : LDBL - NGT]], axis=1)
    boxes_r = boxes.reshape(B, 4 * NGT)
    bdbl = jnp.concatenate([boxes_r, boxes_r[:, : BDBL - 4 * NGT]], axis=1)
    comp_r = boxes[:, :, (2, 3, 2, 3)].reshape(B, 4 * NGT)
    bcomp = jnp.concatenate([comp_r, comp_r[:, : BDBL - 4 * NGT]], axis=1)
    nu = jnp.pad(noise_u.reshape(-1), (0, NUPAD - TOTAL))
    rl = jnp.pad(rand_labels.astype(jnp.int32).reshape(-1),
                 (0, NUPAD - TOTAL))
    q, lab, refo, tbo = _build_sc()(
        ldbl.reshape(-1), bdbl.reshape(-1), bcomp.reshape(-1),
        nu, rl, box_noise_raw.reshape(-1), table)
    dn_query = q.reshape(B, PERB, HID)
    dn_ref = refo.reshape(B, PERB, 4)
    dn_query_pos = jnp.zeros_like(dn_query)
    dn_target_labels = lab.reshape(B, PERB)
    dn_target_boxes = tbo.reshape(B, PERB, 4)
    return (dn_query, dn_ref, dn_query_pos, dn_target_labels, dn_target_boxes)


# speedup vs baseline: 1.1681x; 1.0001x over previous
import functools

import jax
import jax.numpy as jnp
from jax import lax
from jax.experimental import pallas as pl
from jax.experimental.pallas import tpu as pltpu
from jax.experimental.pallas import tpu_sc as plsc

B = 8
NGT = 50
DN = 100
TOTAL = B * DN * NGT
PERB = DN * NGT
HID = 256
BTOT = 4 * TOTAL

NW = 32
SHARE = 1248
QPAD = 1280
CHUNK = 128
NCH = 10
LAST = SHARE - 9 * CHUNK
BELEM = 4 * (SHARE + 8)
LDBL = 72
BDBL = 400

NUPAD = 40032


@functools.cache
def _build_sc():
    mesh = plsc.VectorSubcoreMesh(core_axis_name="c", subcore_axis_name="s")
    return pl.kernel(
        _sc_body,
        mesh=mesh,
        out_type=(
            jax.ShapeDtypeStruct((B, PERB, HID), jnp.float32),
            jax.ShapeDtypeStruct((TOTAL,), jnp.int32),
            jax.ShapeDtypeStruct((BTOT,), jnp.float32),
            jax.ShapeDtypeStruct((BTOT,), jnp.float32),
        ),
        scratch_types=[
            pltpu.VMEM((LDBL,), jnp.int32),
            pltpu.VMEM((BDBL,), jnp.float32),
            pltpu.VMEM((BDBL,), jnp.float32),
            pltpu.VMEM((QPAD,), jnp.float32),
            pltpu.VMEM((QPAD,), jnp.int32),
            pltpu.VMEM((NCH, CHUNK), jnp.int32),
            pltpu.VMEM((QPAD,), jnp.int32),
            pltpu.VMEM((CHUNK, HID), jnp.float32),
            pltpu.VMEM((CHUNK, HID), jnp.float32),
            pltpu.VMEM((BELEM,), jnp.float32),
            pltpu.VMEM((BELEM,), jnp.float32),
            pltpu.VMEM((BELEM,), jnp.float32),
            pltpu.SemaphoreType.DMA,
            pltpu.SemaphoreType.DMA,
            pltpu.SemaphoreType.DMA,
            pltpu.SemaphoreType.DMA,
            pltpu.SemaphoreType.DMA,
        ],
    )


def _sc_body(ldbl_h, bdbl_h, bcomp_h, nu_h, rl_h, bnr_h, table_h,
             q_out, lab_out, ref_out, tbox_out,
             ldbl_v, bdbl_v, bcomp_v, nu_v, rl_v, idx_v, lab_v,
             rows0_v, rows1_v, bnr_v, refo_v, tbo_v,
             sem_in, sem_g0, sem_g1, sem_o0, sem_o1):
    wid = lax.axis_index("s") * 2 + lax.axis_index("c")
    b = wid // 4
    lw = lax.rem(wid, 4)
    lrow0 = lw * SHARE
    qrow0 = b * PERB + lrow0
    bbase = 4 * qrow0
    qphase0 = lax.rem(qrow0, NGT)
    bphase0 = lax.rem(bbase, 4 * NGT)
    is_big = lw == 3

    ins = [
        pltpu.async_copy(ldbl_h.at[pl.ds(b * LDBL, LDBL)], ldbl_v, sem_in),
        pltpu.async_copy(bdbl_h.at[pl.ds(b * BDBL, BDBL)], bdbl_v, sem_in),
        pltpu.async_copy(bcomp_h.at[pl.ds(b * BDBL, BDBL)], bcomp_v, sem_in),
        pltpu.async_copy(nu_h.at[pl.ds(qrow0, QPAD)], nu_v, sem_in),
        pltpu.async_copy(rl_h.at[pl.ds(qrow0, QPAD)], rl_v, sem_in),
        pltpu.async_copy(bnr_h.at[pl.ds(bbase, 4 * SHARE)],
                         bnr_v.at[pl.ds(0, 4 * SHARE)], sem_in),
    ]
    for c in ins:
        c.wait()

    @pl.when(is_big)
    def _():
        pltpu.sync_copy(bnr_h.at[pl.ds(bbase + 4 * SHARE, 32)],
                        bnr_v.at[pl.ds(4 * SHARE, 32)])

    rows = (rows0_v, rows1_v)
    gsem = (sem_g0, sem_g1)
    osem = (sem_o0, sem_o1)
    gh = [None, None]
    oh = [None, None]

    for j in range(NCH):
        def grp(k, carry, j=j):
            off = j * CHUNK + k * 16
            phase = lax.rem(qphase0 + off, NGT)
            rep = ldbl_v[pl.ds(phase, 16)]
            lab_v[pl.ds(off, 16)] = rep
            nu = nu_v[pl.ds(off, 16)]
            rl = rl_v[pl.ds(off, 16)]
            idx_v[j, pl.ds(k * 16, 16)] = jnp.where(nu < 0.5, rl, rep)
            return carry
        lax.fori_loop(0, CHUNK // 16, grp, 0)
        s = j & 1
        if oh[s] is not None:
            oh[s].wait()
            oh[s] = None
        gh[s] = pltpu.async_copy(table_h.at[idx_v.at[j]], rows[s], gsem[s])
        if j >= 1:
            p = (j - 1) & 1
            gh[p].wait()
            gh[p] = None
            n = CHUNK if j - 1 < NCH - 1 else LAST
            oh[p] = pltpu.async_copy(
                rows[p].at[pl.ds(0, n)],
                q_out.at[b, pl.ds(lrow0 + (j - 1) * CHUNK, n)], osem[p])

    s = (NCH - 1) & 1
    gh[s].wait()
    oh[s] = pltpu.async_copy(
        rows[s].at[pl.ds(0, LAST)],
        q_out.at[b, pl.ds(lrow0 + (NCH - 1) * CHUNK, LAST)], osem[s])

    @pl.when(is_big)
    def _():
        pltpu.sync_copy(
            rows[s].at[pl.ds(LAST, 8)],
            q_out.at[b, pl.ds(lrow0 + (NCH - 1) * CHUNK + LAST, 8)])

    def bgrp(i, carry):
        off = i * 16
        bphase = lax.rem(bphase0 + off, 4 * NGT)
        rb = bdbl_v[pl.ds(bphase, 16)]
        cb = bcomp_v[pl.ds(bphase, 16)]
        bn = bnr_v[pl.ds(off, 16)] * 0.8 - 0.4
        out = jnp.minimum(jnp.maximum(rb + bn * cb, 0.0), 1.0)
        refo_v[pl.ds(off, 16)] = out
        tbo_v[pl.ds(off, 16)] = rb
        return carry
    lax.fori_loop(0, BELEM // 16, bgrp, 0)

    pltpu.sync_copy(lab_v.at[pl.ds(0, SHARE)],
                    lab_out.at[pl.ds(qrow0, SHARE)])
    pltpu.sync_copy(refo_v.at[pl.ds(0, 4 * SHARE)],
                    ref_out.at[pl.ds(bbase, 4 * SHARE)])
    pltpu.sync_copy(tbo_v.at[pl.ds(0, 4 * SHARE)],
                    tbox_out.at[pl.ds(bbase, 4 * SHARE)])

    @pl.when(is_big)
    def _():
        pltpu.sync_copy(lab_v.at[pl.ds(SHARE, 8)],
                        lab_out.at[pl.ds(qrow0 + SHARE, 8)])
        pltpu.sync_copy(refo_v.at[pl.ds(4 * SHARE, 32)],
                        ref_out.at[pl.ds(bbase + 4 * SHARE, 32)])
        pltpu.sync_copy(tbo_v.at[pl.ds(4 * SHARE, 32)],
                        tbox_out.at[pl.ds(bbase + 4 * SHARE, 32)])

    for h in oh:
        if h is not None:
            h.wait()


def kernel(labels, boxes, noise_u, rand_labels, box_noise_raw, table):
    labels = labels.astype(jnp.int32)
    ldbl = jnp.concatenate([labels, labels[:, : LDBL - NGT]], axis=1)
    boxes_r = boxes.reshape(B, 4 * NGT)
    bdbl = jnp.concatenate([boxes_r, boxes_r[:, : BDBL - 4 * NGT]], axis=1)
    comp_r = boxes[:, :, (2, 3, 2, 3)].reshape(B, 4 * NGT)
    bcomp = jnp.concatenate([comp_r, comp_r[:, : BDBL - 4 * NGT]], axis=1)
    nu = jnp.pad(noise_u.reshape(-1), (0, NUPAD - TOTAL))
    rl = jnp.pad(rand_labels.astype(jnp.int32).reshape(-1),
                 (0, NUPAD - TOTAL))
    q, lab, refo, tbo = _build_sc()(
        ldbl.reshape(-1), bdbl.reshape(-1), bcomp.reshape(-1),
        nu, rl, box_noise_raw.reshape(-1), table)
    dn_query = q
    dn_ref = refo.reshape(B, PERB, 4)
    dn_query_pos = jnp.zeros_like(dn_query)
    dn_target_labels = lab.reshape(B, PERB)
    dn_target_boxes = tbo.reshape(B, PERB, 4)
    return (dn_query, dn_ref, dn_query_pos, dn_target_labels, dn_target_boxes)


# speedup vs baseline: 1.5223x; 1.3032x over previous
import functools

import jax
import jax.numpy as jnp
from jax import lax
from jax.experimental import pallas as pl
from jax.experimental.pallas import tpu as pltpu
from jax.experimental.pallas import tpu_sc as plsc

B = 8
NGT = 50
DN = 100
TOTAL = B * DN * NGT
PERB = DN * NGT
HID = 256

NW = 32
SHARE = 1248
QPAD = 1280
CHUNK = 128
NCH = 10
LAST = SHARE - 9 * CHUNK
LDBL = 72
NUPAD = 40032

BOX_BLK = 1000


@functools.cache
def _build_sc():
    mesh = plsc.VectorSubcoreMesh(core_axis_name="c", subcore_axis_name="s")
    return pl.kernel(
        _sc_body,
        mesh=mesh,
        out_type=(
            jax.ShapeDtypeStruct((B, PERB, HID), jnp.float32),
            jax.ShapeDtypeStruct((TOTAL,), jnp.int32),
        ),
        scratch_types=[
            pltpu.VMEM((LDBL,), jnp.int32),
            pltpu.VMEM((QPAD,), jnp.float32),
            pltpu.VMEM((QPAD,), jnp.int32),
            pltpu.VMEM((NCH, CHUNK), jnp.int32),
            pltpu.VMEM((QPAD,), jnp.int32),
            pltpu.VMEM((CHUNK, HID), jnp.float32),
            pltpu.VMEM((CHUNK, HID), jnp.float32),
            pltpu.SemaphoreType.DMA,
            pltpu.SemaphoreType.DMA,
            pltpu.SemaphoreType.DMA,
            pltpu.SemaphoreType.DMA,
            pltpu.SemaphoreType.DMA,
        ],
    )


def _sc_body(ldbl_h, nu_h, rl_h, table_h,
             q_out, lab_out,
             ldbl_v, nu_v, rl_v, idx_v, lab_v, rows0_v, rows1_v,
             sem_in, sem_g0, sem_g1, sem_o0, sem_o1):
    wid = lax.axis_index("s") * 2 + lax.axis_index("c")
    b = wid // 4
    lw = lax.rem(wid, 4)
    lrow0 = lw * SHARE
    qrow0 = b * PERB + lrow0
    qphase0 = lax.rem(qrow0, NGT)
    is_big = lw == 3

    ins = [
        pltpu.async_copy(ldbl_h.at[pl.ds(b * LDBL, LDBL)], ldbl_v, sem_in),
        pltpu.async_copy(nu_h.at[pl.ds(qrow0, QPAD)], nu_v, sem_in),
        pltpu.async_copy(rl_h.at[pl.ds(qrow0, QPAD)], rl_v, sem_in),
    ]
    for c in ins:
        c.wait()

    rows = (rows0_v, rows1_v)
    gsem = (sem_g0, sem_g1)
    osem = (sem_o0, sem_o1)
    gh = [None, None]
    oh = [None, None]

    for j in range(NCH):
        def grp(k, carry, j=j):
            off = j * CHUNK + k * 16
            phase = lax.rem(qphase0 + off, NGT)
            rep = ldbl_v[pl.ds(phase, 16)]
            lab_v[pl.ds(off, 16)] = rep
            nu = nu_v[pl.ds(off, 16)]
            rl = rl_v[pl.ds(off, 16)]
            idx_v[j, pl.ds(k * 16, 16)] = jnp.where(nu < 0.5, rl, rep)
            return carry
        lax.fori_loop(0, CHUNK // 16, grp, 0)
        s = j & 1
        if oh[s] is not None:
            oh[s].wait()
            oh[s] = None
        gh[s] = pltpu.async_copy(table_h.at[idx_v.at[j]], rows[s], gsem[s])
        if j >= 1:
            p = (j - 1) & 1
            gh[p].wait()
            gh[p] = None
            oh[p] = pltpu.async_copy(
                rows[p].at[pl.ds(0, CHUNK)],
                q_out.at[b, pl.ds(lrow0 + (j - 1) * CHUNK, CHUNK)], osem[p])

    s = (NCH - 1) & 1
    gh[s].wait()
    oh[s] = pltpu.async_copy(
        rows[s].at[pl.ds(0, LAST)],
        q_out.at[b, pl.ds(lrow0 + (NCH - 1) * CHUNK, LAST)], osem[s])

    @pl.when(is_big)
    def _():
        pltpu.sync_copy(
            rows[s].at[pl.ds(LAST, 8)],
            q_out.at[b, pl.ds(lrow0 + (NCH - 1) * CHUNK + LAST, 8)])

    pltpu.sync_copy(lab_v.at[pl.ds(0, SHARE)],
                    lab_out.at[pl.ds(qrow0, SHARE)])

    @pl.when(is_big)
    def _():
        pltpu.sync_copy(lab_v.at[pl.ds(SHARE, 8)],
                        lab_out.at[pl.ds(qrow0 + SHARE, 8)])

    for h in oh:
        if h is not None:
            h.wait()


def _box_body(bnr_ref, rep_ref, comp_ref, refo_ref, tbo_ref):
    rep = rep_ref[...]
    bn = bnr_ref[...] * 0.8 - 0.4
    out = jnp.minimum(jnp.maximum(rep + bn * comp_ref[...], 0.0), 1.0)
    refo_ref[...] = out
    tbo_ref[...] = rep


@functools.cache
def _build_box():
    spec_io = pl.BlockSpec((1, BOX_BLK, 4), lambda b, j: (b, j, 0))
    spec_pat = pl.BlockSpec((1, BOX_BLK, 4), lambda b, j: (b, 0, 0))
    return pl.pallas_call(
        _box_body,
        grid=(B, PERB // BOX_BLK),
        in_specs=[spec_io, spec_pat, spec_pat],
        out_specs=[spec_io, spec_io],
        out_shape=[jax.ShapeDtypeStruct((B, PERB, 4), jnp.float32)] * 2,
    )


def kernel(labels, boxes, noise_u, rand_labels, box_noise_raw, table):
    labels = labels.astype(jnp.int32)
    ldbl = jnp.concatenate([labels, labels[:, : LDBL - NGT]], axis=1)
    nu = jnp.pad(noise_u.reshape(-1), (0, NUPAD - TOTAL))
    rl = jnp.pad(rand_labels.astype(jnp.int32).reshape(-1),
                 (0, NUPAD - TOTAL))
    rep_pre = jnp.tile(boxes, (1, BOX_BLK // NGT, 1))
    comp_pre = jnp.tile(boxes[:, :, (2, 3, 2, 3)], (1, BOX_BLK // NGT, 1))
    q, lab = _build_sc()(ldbl.reshape(-1), nu, rl, table)
    dn_ref, dn_target_boxes = _build_box()(box_noise_raw, rep_pre, comp_pre)
    dn_query_pos = jnp.zeros((B, PERB, HID), jnp.float32)
    dn_target_labels = lab.reshape(B, PERB)
    return (q, dn_ref, dn_query_pos, dn_target_labels, dn_target_boxes)


# speedup vs baseline: 2.6289x; 1.7269x over previous
import functools

import jax
import jax.numpy as jnp
from jax import lax
from jax.experimental import pallas as pl
from jax.experimental.pallas import tpu as pltpu
from jax.experimental.pallas import tpu_sc as plsc

B = 8
NGT = 50
DN = 100
TOTAL = B * DN * NGT
PERB = DN * NGT
HID = 256

NW = 32
SHARE = 1248
QPAD = 1280
CHUNK = 128
NCH = 10
LAST = SHARE - 9 * CHUNK
LDBL = 72
NUPAD = 40032

BOX_BLK = 1000


@functools.cache
def _build_sc():
    mesh = plsc.VectorSubcoreMesh(core_axis_name="c", subcore_axis_name="s")
    return pl.kernel(
        _sc_body,
        mesh=mesh,
        out_type=(
            jax.ShapeDtypeStruct((B, PERB, HID), jnp.float32),
            jax.ShapeDtypeStruct((TOTAL,), jnp.int32),
        ),
        scratch_types=[
            pltpu.VMEM((LDBL,), jnp.int32),
            pltpu.VMEM((QPAD,), jnp.float32),
            pltpu.VMEM((QPAD,), jnp.int32),
            pltpu.VMEM((NCH, CHUNK), jnp.int32),
            pltpu.VMEM((QPAD,), jnp.int32),
            pltpu.VMEM((CHUNK, HID), jnp.float32),
            pltpu.VMEM((CHUNK, HID), jnp.float32),
            pltpu.SemaphoreType.DMA,
            pltpu.SemaphoreType.DMA,
            pltpu.SemaphoreType.DMA,
            pltpu.SemaphoreType.DMA,
            pltpu.SemaphoreType.DMA,
        ],
    )


def _sc_body(ldbl_h, nu_h, rl_h, table_h,
             q_out, lab_out,
             ldbl_v, nu_v, rl_v, idx_v, lab_v, rows0_v, rows1_v,
             sem_in, sem_g0, sem_g1, sem_o0, sem_o1):
    wid = lax.axis_index("s") * 2 + lax.axis_index("c")
    b = wid // 4
    lw = lax.rem(wid, 4)
    lrow0 = lw * SHARE
    qrow0 = b * PERB + lrow0
    qphase0 = lax.rem(qrow0, NGT)
    is_big = lw == 3

    ins = [
        pltpu.async_copy(ldbl_h.at[pl.ds(b * LDBL, LDBL)], ldbl_v, sem_in),
        pltpu.async_copy(nu_h.at[pl.ds(qrow0, QPAD)], nu_v, sem_in),
        pltpu.async_copy(rl_h.at[pl.ds(qrow0, QPAD)], rl_v, sem_in),
    ]
    for c in ins:
        c.wait()

    rows = (rows0_v, rows1_v)
    gsem = (sem_g0, sem_g1)
    osem = (sem_o0, sem_o1)
    gh = [None, None]
    oh = [None, None]

    for j in range(NCH):
        def grp(k, carry, j=j):
            off = j * CHUNK + k * 16
            phase = lax.rem(qphase0 + off, NGT)
            rep = ldbl_v[pl.ds(phase, 16)]
            lab_v[pl.ds(off, 16)] = rep
            nu = nu_v[pl.ds(off, 16)]
            rl = rl_v[pl.ds(off, 16)]
            idx_v[j, pl.ds(k * 16, 16)] = jnp.where(nu < 0.5, rl, rep)
            return carry
        lax.fori_loop(0, CHUNK // 16, grp, 0)
        s = j & 1
        if oh[s] is not None:
            oh[s].wait()
            oh[s] = None
        gh[s] = pltpu.async_copy(table_h.at[idx_v.at[j]], rows[s], gsem[s])
        if j >= 1:
            p = (j - 1) & 1
            gh[p].wait()
            gh[p] = None
            oh[p] = pltpu.async_copy(
                rows[p].at[pl.ds(0, CHUNK)],
                q_out.at[b, pl.ds(lrow0 + (j - 1) * CHUNK, CHUNK)], osem[p])

    s = (NCH - 1) & 1
    gh[s].wait()
    oh[s] = pltpu.async_copy(
        rows[s].at[pl.ds(0, LAST)],
        q_out.at[b, pl.ds(lrow0 + (NCH - 1) * CHUNK, LAST)], osem[s])

    @pl.when(is_big)
    def _():
        pltpu.sync_copy(
            rows[s].at[pl.ds(LAST, 8)],
            q_out.at[b, pl.ds(lrow0 + (NCH - 1) * CHUNK + LAST, 8)])

    pltpu.sync_copy(lab_v.at[pl.ds(0, SHARE)],
                    lab_out.at[pl.ds(qrow0, SHARE)])

    @pl.when(is_big)
    def _():
        pltpu.sync_copy(lab_v.at[pl.ds(SHARE, 8)],
                        lab_out.at[pl.ds(qrow0 + SHARE, 8)])

    for h in oh:
        if h is not None:
            h.wait()


def _box_body(bnr_ref, rep_ref, comp_ref, refo_ref, tbo_ref):
    reps = PERB // BOX_BLK
    rep = jnp.concatenate([rep_ref[...]] * reps, axis=2)
    comp = jnp.concatenate([comp_ref[...]] * reps, axis=2)
    bn = bnr_ref[...] * 0.8 - 0.4
    out = jnp.minimum(jnp.maximum(rep + bn * comp, 0.0), 1.0)
    refo_ref[...] = out
    tbo_ref[...] = rep


@functools.cache
def _build_box():
    return pl.pallas_call(
        _box_body,
        out_shape=[jax.ShapeDtypeStruct((B, 4, PERB), jnp.float32)] * 2,
    )


def kernel(labels, boxes, noise_u, rand_labels, box_noise_raw, table):
    labels = labels.astype(jnp.int32)
    ldbl = jnp.concatenate([labels, labels[:, : LDBL - NGT]], axis=1)
    nu = jnp.pad(noise_u.reshape(-1), (0, NUPAD - TOTAL))
    rl = jnp.pad(rand_labels.astype(jnp.int32).reshape(-1),
                 (0, NUPAD - TOTAL))
    boxes_t = jnp.transpose(boxes, (0, 2, 1))
    comp_t = jnp.concatenate([boxes_t[:, 2:4], boxes_t[:, 2:4]], axis=1)
    rep_pre = jnp.tile(boxes_t, (1, 1, BOX_BLK // NGT))
    comp_pre = jnp.tile(comp_t, (1, 1, BOX_BLK // NGT))
    bnr_t = jnp.transpose(box_noise_raw, (0, 2, 1))
    q, lab = _build_sc()(ldbl.reshape(-1), nu, rl, table)
    refo_t, tbo_t = _build_box()(bnr_t, rep_pre, comp_pre)
    dn_ref = jnp.transpose(refo_t, (0, 2, 1))
    dn_target_boxes = jnp.transpose(tbo_t, (0, 2, 1))
    dn_query_pos = jnp.zeros((B, PERB, HID), jnp.float32)
    dn_target_labels = lab.reshape(B, PERB)
    return (q, dn_ref, dn_query_pos, dn_target_labels, dn_target_boxes)


# speedup vs baseline: 2.7009x; 1.0274x over previous
import functools

import jax
import jax.numpy as jnp
from jax import lax
from jax.experimental import pallas as pl
from jax.experimental.pallas import tpu as pltpu
from jax.experimental.pallas import tpu_sc as plsc

B = 8
NGT = 50
DN = 100
TOTAL = B * DN * NGT
PERB = DN * NGT
HID = 256

NW = 32
SHARE = 1248
QPAD = 1280
CHUNK = 128
NCH = 10
LAST = SHARE - 9 * CHUNK
LDBL = 72
NUPAD = 40032

BOX_BLK = 1000


@functools.cache
def _build_sc():
    mesh = plsc.VectorSubcoreMesh(core_axis_name="c", subcore_axis_name="s")
    return pl.kernel(
        _sc_body,
        mesh=mesh,
        out_type=(
            jax.ShapeDtypeStruct((B, PERB, HID), jnp.float32),
            jax.ShapeDtypeStruct((TOTAL,), jnp.int32),
        ),
        scratch_types=[
            pltpu.VMEM((LDBL,), jnp.int32),
            pltpu.VMEM((QPAD,), jnp.float32),
            pltpu.VMEM((QPAD,), jnp.int32),
            pltpu.VMEM((NCH, CHUNK), jnp.int32),
            pltpu.VMEM((QPAD,), jnp.int32),
            pltpu.VMEM((CHUNK, HID), jnp.float32),
            pltpu.VMEM((CHUNK, HID), jnp.float32),
            pltpu.SemaphoreType.DMA,
            pltpu.SemaphoreType.DMA,
            pltpu.SemaphoreType.DMA,
            pltpu.SemaphoreType.DMA,
            pltpu.SemaphoreType.DMA,
        ],
    )


def _sc_body(ldbl_h, nu_h, rl_h, table_h,
             q_out, lab_out,
             ldbl_v, nu_v, rl_v, idx_v, lab_v, rows0_v, rows1_v,
             sem_in, sem_g0, sem_g1, sem_o0, sem_o1):
    wid = lax.axis_index("s") * 2 + lax.axis_index("c")
    b = wid // 4
    lw = lax.rem(wid, 4)
    lrow0 = lw * SHARE
    qrow0 = b * PERB + lrow0
    qphase0 = lax.rem(qrow0, NGT)
    is_big = lw == 3

    ins = [
        pltpu.async_copy(ldbl_h.at[pl.ds(b * LDBL, LDBL)], ldbl_v, sem_in),
        pltpu.async_copy(nu_h.at[pl.ds(qrow0, QPAD)], nu_v, sem_in),
        pltpu.async_copy(rl_h.at[pl.ds(qrow0, QPAD)], rl_v, sem_in),
    ]
    for c in ins:
        c.wait()

    rows = (rows0_v, rows1_v)
    gsem = (sem_g0, sem_g1)
    osem = (sem_o0, sem_o1)
    gh = [None, None]
    oh = [None, None]

    for j in range(NCH):
        def grp(k, carry, j=j):
            off = j * CHUNK + k * 16
            phase = lax.rem(qphase0 + off, NGT)
            rep = ldbl_v[pl.ds(phase, 16)]
            lab_v[pl.ds(off, 16)] = rep
            nu = nu_v[pl.ds(off, 16)]
            rl = rl_v[pl.ds(off, 16)]
            idx_v[j, pl.ds(k * 16, 16)] = jnp.where(nu < 0.5, rl, rep)
            return carry
        lax.fori_loop(0, CHUNK // 16, grp, 0)
        s = j & 1
        if oh[s] is not None:
            oh[s].wait()
            oh[s] = None
        gh[s] = pltpu.async_copy(table_h.at[idx_v.at[j]], rows[s], gsem[s])
        if j >= 1:
            p = (j - 1) & 1
            gh[p].wait()
            gh[p] = None
            oh[p] = pltpu.async_copy(
                rows[p].at[pl.ds(0, CHUNK)],
                q_out.at[b, pl.ds(lrow0 + (j - 1) * CHUNK, CHUNK)], osem[p])

    s = (NCH - 1) & 1
    gh[s].wait()
    oh[s] = pltpu.async_copy(
        rows[s].at[pl.ds(0, LAST)],
        q_out.at[b, pl.ds(lrow0 + (NCH - 1) * CHUNK, LAST)], osem[s])

    @pl.when(is_big)
    def _():
        pltpu.sync_copy(
            rows[s].at[pl.ds(LAST, 8)],
            q_out.at[b, pl.ds(lrow0 + (NCH - 1) * CHUNK + LAST, 8)])

    pltpu.sync_copy(lab_v.at[pl.ds(0, SHARE)],
                    lab_out.at[pl.ds(qrow0, SHARE)])

    @pl.when(is_big)
    def _():
        pltpu.sync_copy(lab_v.at[pl.ds(SHARE, 8)],
                        lab_out.at[pl.ds(qrow0 + SHARE, 8)])

    for h in oh:
        if h is not None:
            h.wait()


def _box_body(bnr_ref, rep_ref, comp_ref, refo_ref, tbo_ref, qpos_ref):
    reps = PERB // BOX_BLK
    rep = jnp.concatenate([rep_ref[...]] * reps, axis=2)
    comp = jnp.concatenate([comp_ref[...]] * reps, axis=2)
    bn = bnr_ref[...] * 0.8 - 0.4
    out = jnp.minimum(jnp.maximum(rep + bn * comp, 0.0), 1.0)
    refo_ref[...] = out
    tbo_ref[...] = rep
    qpos_ref[...] = jnp.zeros_like(qpos_ref)


@functools.cache
def _build_box():
    return pl.pallas_call(
        _box_body,
        grid=(B,),
        in_specs=[
            pl.BlockSpec((1, 4, PERB), lambda b: (b, 0, 0)),
            pl.BlockSpec((1, 4, BOX_BLK), lambda b: (b, 0, 0)),
            pl.BlockSpec((1, 4, BOX_BLK), lambda b: (b, 0, 0)),
        ],
        out_specs=[
            pl.BlockSpec((1, 4, PERB), lambda b: (b, 0, 0)),
            pl.BlockSpec((1, 4, PERB), lambda b: (b, 0, 0)),
            pl.BlockSpec((1, PERB, HID), lambda b: (b, 0, 0)),
        ],
        out_shape=[
            jax.ShapeDtypeStruct((B, 4, PERB), jnp.float32),
            jax.ShapeDtypeStruct((B, 4, PERB), jnp.float32),
            jax.ShapeDtypeStruct((B, PERB, HID), jnp.float32),
        ],
    )


def kernel(labels, boxes, noise_u, rand_labels, box_noise_raw, table):
    labels = labels.astype(jnp.int32)
    ldbl = jnp.concatenate([labels, labels[:, : LDBL - NGT]], axis=1)
    nu = jnp.pad(noise_u.reshape(-1), (0, NUPAD - TOTAL))
    rl = jnp.pad(rand_labels.astype(jnp.int32).reshape(-1),
                 (0, NUPAD - TOTAL))
    boxes_t = jnp.transpose(boxes, (0, 2, 1))
    comp_t = jnp.concatenate([boxes_t[:, 2:4], boxes_t[:, 2:4]], axis=1)
    rep_pre = jnp.tile(boxes_t, (1, 1, BOX_BLK // NGT))
    comp_pre = jnp.tile(comp_t, (1, 1, BOX_BLK // NGT))
    bnr_t = jnp.transpose(box_noise_raw, (0, 2, 1))
    q, lab = _build_sc()(ldbl.reshape(-1), nu, rl, table)
    refo_t, tbo_t, dn_query_pos = _build_box()(bnr_t, rep_pre, comp_pre)
    dn_ref = jnp.transpose(refo_t, (0, 2, 1))
    dn_target_boxes = jnp.transpose(tbo_t, (0, 2, 1))
    dn_target_labels = lab.reshape(B, PERB)
    return (q, dn_ref, dn_query_pos, dn_target_labels, dn_target_boxes)
